# baseline (speedup 1.0000x reference)
import functools

import jax
import jax.numpy as jnp
from jax import lax
from jax.experimental import pallas as pl
from jax.experimental.pallas import tpu as pltpu
from jax.experimental.pallas import tpu_sc as plsc

_VOCAB = 1000
_DIM = 128
_PDIM = _DIM // 2
_FEATURES = 26
_SC_FIELDS = 20
_BATCH = 16384

_NUM_CORES = 2
_NUM_SUBCORES = 16
_NUM_WORKERS = _NUM_CORES * _NUM_SUBCORES
_ROWS_PER_WORKER = _BATCH // _NUM_WORKERS
_CHUNK = 256
_NUM_CHUNKS = _ROWS_PER_WORKER // _CHUNK
_LANES = 16

_mesh = plsc.VectorSubcoreMesh(core_axis_name="c", subcore_axis_name="s")


@functools.partial(
    pl.kernel,
    out_type=jax.ShapeDtypeStruct((_BATCH, _DIM), jnp.float32),
    mesh=_mesh,
    scratch_types=[
        pltpu.VMEM((_SC_FIELDS, _NUM_CHUNKS, _CHUNK), jnp.int32),
        pltpu.VMEM((_CHUNK, _PDIM), jnp.float32),
        pltpu.VMEM((_CHUNK, _PDIM), jnp.float32),
        pltpu.VMEM((_CHUNK, _PDIM), jnp.int32),
        pltpu.VMEM((_CHUNK, _PDIM), jnp.int32),
        pltpu.VMEM((_CHUNK, _PDIM), jnp.int32),
        pltpu.VMEM((_CHUNK, _PDIM), jnp.int32),
        pltpu.SemaphoreType.DMA,
        pltpu.SemaphoreType.DMA,
        pltpu.SemaphoreType.DMA,
        pltpu.SemaphoreType.DMA,
        pltpu.SemaphoreType.DMA,
    ],
    compiler_params=pltpu.CompilerParams(use_tc_tiling_on_sc=False),
)
def _embed_sum(w_hbm, idx_hbm, out_hbm, idx_v, acc_lo, acc_hi,
               st0, st1, st2, st3, sem_o, sem0, sem1, sem2, sem3):
    wid = lax.axis_index("s") * _NUM_CORES + lax.axis_index("c")
    base = wid * _ROWS_PER_WORKER
    sts = [st0, st1, st2, st3]
    sems = [sem0, sem1, sem2, sem3]
    pltpu.sync_copy(idx_hbm.at[:, wid], idx_v)

    _MASK = jnp.int32(-65536)

    def accumulate_pair(sa, sb, first):
        @plsc.parallel_loop(0, _CHUNK, 1, unroll=2)
        def _(r):
            for c in range(_PDIM // _LANES):
                sl = pl.ds(c * _LANES, _LANES)
                va = sa[r, sl]
                vb = sb[r, sl]
                lo = (lax.bitcast_convert_type(va << 16, jnp.float32)
                      + lax.bitcast_convert_type(vb << 16, jnp.float32))
                hi = (lax.bitcast_convert_type(va & _MASK, jnp.float32)
                      + lax.bitcast_convert_type(vb & _MASK, jnp.float32))
                if first:
                    acc_lo[r, sl] = lo
                    acc_hi[r, sl] = hi
                else:
                    plsc.addupdate(acc_lo.at[r, sl], lo)
                    plsc.addupdate(acc_hi.at[r, sl], hi)

    def chunk_body(ch, carry):
        descs = [
            pltpu.async_copy(w_hbm.at[idx_v.at[i, ch]], sts[i], sems[i])
            for i in range(4)
        ]
        for grp in range(_SC_FIELDS // 2):
            i0, i1 = 2 * grp, 2 * grp + 1
            descs[i0 % 4].wait()
            descs[i1 % 4].wait()
            accumulate_pair(sts[i0 % 4], sts[i1 % 4], first=(grp == 0))
            if i1 + 4 < _SC_FIELDS:
                descs[i0 % 4] = pltpu.async_copy(
                    w_hbm.at[idx_v.at[i0 + 4, ch]], sts[i0 % 4],
                    sems[i0 % 4])
                descs[i1 % 4] = pltpu.async_copy(
                    w_hbm.at[idx_v.at[i1 + 4, ch]], sts[i1 % 4],
                    sems[i1 % 4])
        rows = pl.ds(base + ch * _CHUNK, _CHUNK)
        c_lo = pltpu.async_copy(
            acc_lo, out_hbm.at[rows, pl.ds(0, _PDIM)], sem_o)
        pltpu.sync_copy(acc_hi, out_hbm.at[rows, pl.ds(_PDIM, _PDIM)])
        c_lo.wait()
        return carry

    lax.fori_loop(0, _NUM_CHUNKS, chunk_body, 0)


@jax.jit
def kernel(x, W):
    xi = x.astype(jnp.int32)
    offs = jnp.arange(_SC_FIELDS, dtype=jnp.int32) * _VOCAB
    idx = (xi[:, :_SC_FIELDS] + offs[None, :]).T.reshape(
        _SC_FIELDS, _NUM_WORKERS, _NUM_CHUNKS, _CHUNK)
    bits = lax.bitcast_convert_type(W[:_SC_FIELDS], jnp.int32)
    rnd = bits + jnp.int32(0x7FFF) + ((bits >> 16) & 1)
    b16 = (rnd >> 16) & jnp.int32(0xFFFF)
    w_pack = (b16[..., :_PDIM] | (b16[..., _PDIM:] << 16)).reshape(
        _SC_FIELDS * _VOCAB, _PDIM)
    sc_out = _embed_sum(w_pack, idx)
    vocab_iota = jnp.arange(_VOCAB, dtype=jnp.int32)
    tc_out = jnp.zeros((_BATCH, _DIM), jnp.float32)
    for i in range(_SC_FIELDS, _FEATURES):
        oh = (xi[:, i, None] == vocab_iota[None, :]).astype(jnp.bfloat16)
        tc_out = tc_out + jnp.dot(
            oh, W[i].astype(jnp.bfloat16),
            preferred_element_type=jnp.float32)
    return sc_out + tc_out


# speedup vs baseline: 1.1723x; 1.1723x over previous
import functools

import jax
import jax.numpy as jnp
from jax import lax
from jax.experimental import pallas as pl
from jax.experimental.pallas import tpu as pltpu
from jax.experimental.pallas import tpu_sc as plsc

_VOCAB = 1000
_DIM = 128
_PDIM = _DIM // 2
_FEATURES = 26
_BATCH = 16384

_NUM_CORES = 2
_NUM_SUBCORES = 16
_FPC = _FEATURES // _NUM_CORES
_TROWS = _FPC * _VOCAB
_TPAD = 13312
_STRIPE = _TPAD // _NUM_SUBCORES
_ROWS_PER_SUBCORE = _BATCH // _NUM_SUBCORES
_CHUNK = 128
_NUM_CHUNKS = _ROWS_PER_SUBCORE // _CHUNK
_LANES = 16

_mesh = plsc.VectorSubcoreMesh(core_axis_name="c", subcore_axis_name="s")


@functools.partial(
    pl.kernel,
    out_type=jax.ShapeDtypeStruct((_NUM_CORES, _BATCH, _DIM), jnp.float32),
    mesh=_mesh,
    scratch_types=[
        pltpu.VMEM_SHARED((_TPAD, _PDIM), jnp.int32),
        pltpu.VMEM((_FPC, _NUM_CHUNKS, _CHUNK), jnp.int32),
        pltpu.VMEM((_CHUNK, _PDIM), jnp.float32),
        pltpu.VMEM((_CHUNK, _PDIM), jnp.float32),
        pltpu.VMEM((_CHUNK, _PDIM), jnp.int32),
        pltpu.VMEM((_CHUNK, _PDIM), jnp.int32),
        pltpu.VMEM((_CHUNK, _PDIM), jnp.int32),
        pltpu.VMEM((_CHUNK, _PDIM), jnp.int32),
        pltpu.SemaphoreType.DMA,
        pltpu.SemaphoreType.DMA,
        pltpu.SemaphoreType.DMA,
        pltpu.SemaphoreType.DMA,
        pltpu.SemaphoreType.DMA,
    ],
    compiler_params=pltpu.CompilerParams(use_tc_tiling_on_sc=False),
)
def _embed_sum(w_hbm, idx_hbm, out_hbm, w_sh, idx_v, acc_lo, acc_hi,
               st0, st1, st2, st3, sem_o, sem0, sem1, sem2, sem3):
    cid = lax.axis_index("c")
    sid = lax.axis_index("s")
    base = sid * _ROWS_PER_SUBCORE
    sts = [st0, st1, st2, st3]
    sems = [sem0, sem1, sem2, sem3]
    rows_sl = pl.ds(sid * _STRIPE, _STRIPE)
    pltpu.sync_copy(w_hbm.at[cid, rows_sl], w_sh.at[rows_sl])
    pltpu.sync_copy(idx_hbm.at[cid, :, sid], idx_v)
    plsc.subcore_barrier()

    _MASK = jnp.int32(-65536)

    def accumulate_pair(sa, sb, first):
        @plsc.parallel_loop(0, _CHUNK, 1, unroll=2)
        def _(r):
            for c in range(_PDIM // _LANES):
                sl = pl.ds(c * _LANES, _LANES)
                va = sa[r, sl]
                vb = sb[r, sl]
                lo = (lax.bitcast_convert_type(va << 16, jnp.float32)
                      + lax.bitcast_convert_type(vb << 16, jnp.float32))
                hi = (lax.bitcast_convert_type(va & _MASK, jnp.float32)
                      + lax.bitcast_convert_type(vb & _MASK, jnp.float32))
                if first:
                    acc_lo[r, sl] = lo
                    acc_hi[r, sl] = hi
                else:
                    plsc.addupdate(acc_lo.at[r, sl], lo)
                    plsc.addupdate(acc_hi.at[r, sl], hi)

    def accumulate_one(sa):
        @plsc.parallel_loop(0, _CHUNK, 1, unroll=2)
        def _(r):
            for c in range(_PDIM // _LANES):
                sl = pl.ds(c * _LANES, _LANES)
                va = sa[r, sl]
                plsc.addupdate(
                    acc_lo.at[r, sl],
                    lax.bitcast_convert_type(va << 16, jnp.float32))
                plsc.addupdate(
                    acc_hi.at[r, sl],
                    lax.bitcast_convert_type(va & _MASK, jnp.float32))

    def chunk_body(ch, carry):
        descs = [
            pltpu.async_copy(w_sh.at[idx_v.at[i, ch]], sts[i], sems[i])
            for i in range(4)
        ]
        for grp in range(_FPC // 2):
            i0, i1 = 2 * grp, 2 * grp + 1
            descs[i0 % 4].wait()
            descs[i1 % 4].wait()
            accumulate_pair(sts[i0 % 4], sts[i1 % 4], first=(grp == 0))
            for nf in (i0 + 4, i1 + 4):
                if nf < _FPC:
                    descs[nf % 4] = pltpu.async_copy(
                        w_sh.at[idx_v.at[nf, ch]], sts[nf % 4],
                        sems[nf % 4])
        descs[(_FPC - 1) % 4].wait()
        accumulate_one(sts[(_FPC - 1) % 4])
        rows = pl.ds(base + ch * _CHUNK, _CHUNK)
        c_lo = pltpu.async_copy(
            acc_lo, out_hbm.at[cid, rows, pl.ds(0, _PDIM)], sem_o)
        pltpu.sync_copy(acc_hi, out_hbm.at[cid, rows, pl.ds(_PDIM, _PDIM)])
        c_lo.wait()
        return carry

    lax.fori_loop(0, _NUM_CHUNKS, chunk_body, 0)


@jax.jit
def kernel(x, W):
    xi = x.astype(jnp.int32)
    offs = (jnp.arange(_FEATURES, dtype=jnp.int32) % _FPC) * _VOCAB
    idx = (xi + offs[None, :]).T.reshape(
        _NUM_CORES, _FPC, _NUM_SUBCORES, _NUM_CHUNKS, _CHUNK)
    bits = lax.bitcast_convert_type(W, jnp.int32)
    rnd = bits + jnp.int32(0x7FFF) + ((bits >> 16) & 1)
    b16 = (rnd >> 16) & jnp.int32(0xFFFF)
    w_pack = (b16[..., :_PDIM] | (b16[..., _PDIM:] << 16)).reshape(
        _NUM_CORES, _TROWS, _PDIM)
    w_pad = jnp.concatenate(
        [w_pack,
         jnp.zeros((_NUM_CORES, _TPAD - _TROWS, _PDIM), jnp.int32)],
        axis=1)
    out = _embed_sum(w_pad, idx)
    return out[0] + out[1]


# speedup vs baseline: 1.2928x; 1.1028x over previous
import functools

import jax
import jax.numpy as jnp
from jax import lax
from jax.experimental import pallas as pl
from jax.experimental.pallas import tpu as pltpu
from jax.experimental.pallas import tpu_sc as plsc

_VOCAB = 1000
_DIM = 128
_PDIM = _DIM // 2
_FEATURES = 26
_BATCH = 16384

_NUM_CORES = 2
_NUM_SUBCORES = 16
_NUM_WORKERS = _NUM_CORES * _NUM_SUBCORES
_ROWS_PER_WORKER = _BATCH // _NUM_WORKERS
_CHUNK = 256
_NUM_CHUNKS = _ROWS_PER_WORKER // _CHUNK
_LANES = 16

_mesh = plsc.VectorSubcoreMesh(core_axis_name="c", subcore_axis_name="s")


@functools.partial(
    pl.kernel,
    out_type=jax.ShapeDtypeStruct((_BATCH, _DIM), jnp.float32),
    mesh=_mesh,
    scratch_types=[
        pltpu.VMEM((_FEATURES, _NUM_CHUNKS, _CHUNK), jnp.int32),
        pltpu.VMEM((_CHUNK, _PDIM), jnp.float32),
        pltpu.VMEM((_CHUNK, _PDIM), jnp.float32),
        pltpu.VMEM((_CHUNK, _PDIM), jnp.int32),
        pltpu.VMEM((_CHUNK, _PDIM), jnp.int32),
        pltpu.VMEM((_CHUNK, _PDIM), jnp.int32),
        pltpu.VMEM((_CHUNK, _PDIM), jnp.int32),
        pltpu.SemaphoreType.DMA,
        pltpu.SemaphoreType.DMA,
        pltpu.SemaphoreType.DMA,
        pltpu.SemaphoreType.DMA,
        pltpu.SemaphoreType.DMA,
    ],
    compiler_params=pltpu.CompilerParams(use_tc_tiling_on_sc=False),
)
def _embed_sum(w_hbm, idx_hbm, out_hbm, idx_v, acc_lo, acc_hi,
               st0, st1, st2, st3, sem_o, sem0, sem1, sem2, sem3):
    wid = lax.axis_index("s") * _NUM_CORES + lax.axis_index("c")
    base = wid * _ROWS_PER_WORKER
    sts = [st0, st1, st2, st3]
    sems = [sem0, sem1, sem2, sem3]
    pltpu.sync_copy(idx_hbm.at[:, wid], idx_v)

    _MASK = jnp.int32(-65536)

    def accumulate_pair(sa, sb, first):
        @plsc.parallel_loop(0, _CHUNK, 1, unroll=2)
        def _(r):
            for c in range(_PDIM // _LANES):
                sl = pl.ds(c * _LANES, _LANES)
                va = sa[r, sl]
                vb = sb[r, sl]
                lo = (lax.bitcast_convert_type(va << 16, jnp.float32)
                      + lax.bitcast_convert_type(vb << 16, jnp.float32))
                hi = (lax.bitcast_convert_type(va & _MASK, jnp.float32)
                      + lax.bitcast_convert_type(vb & _MASK, jnp.float32))
                if first:
                    acc_lo[r, sl] = lo
                    acc_hi[r, sl] = hi
                else:
                    plsc.addupdate(acc_lo.at[r, sl], lo)
                    plsc.addupdate(acc_hi.at[r, sl], hi)

    def chunk_body(ch, carry):
        descs = [
            pltpu.async_copy(w_hbm.at[idx_v.at[i, ch]], sts[i], sems[i])
            for i in range(4)
        ]
        for grp in range(_FEATURES // 2):
            i0, i1 = 2 * grp, 2 * grp + 1
            descs[i0 % 4].wait()
            descs[i1 % 4].wait()
            accumulate_pair(sts[i0 % 4], sts[i1 % 4], first=(grp == 0))
            if i1 + 4 < _FEATURES:
                descs[i0 % 4] = pltpu.async_copy(
                    w_hbm.at[idx_v.at[i0 + 4, ch]], sts[i0 % 4],
                    sems[i0 % 4])
                descs[i1 % 4] = pltpu.async_copy(
                    w_hbm.at[idx_v.at[i1 + 4, ch]], sts[i1 % 4],
                    sems[i1 % 4])
        rows = pl.ds(base + ch * _CHUNK, _CHUNK)
        c_lo = pltpu.async_copy(
            acc_lo, out_hbm.at[rows, pl.ds(0, _PDIM)], sem_o)
        pltpu.sync_copy(acc_hi, out_hbm.at[rows, pl.ds(_PDIM, _PDIM)])
        c_lo.wait()
        return carry

    lax.fori_loop(0, _NUM_CHUNKS, chunk_body, 0)


@jax.jit
def kernel(x, W):
    offs = jnp.arange(_FEATURES, dtype=jnp.int32) * _VOCAB
    idx = (x.astype(jnp.int32) + offs[None, :]).T.reshape(
        _FEATURES, _NUM_WORKERS, _NUM_CHUNKS, _CHUNK)
    bits = lax.bitcast_convert_type(W, jnp.int32)
    rnd = bits + jnp.int32(0x7FFF) + ((bits >> 16) & 1)
    b16 = (rnd >> 16) & jnp.int32(0xFFFF)
    w_pack = (b16[..., :_PDIM] | (b16[..., _PDIM:] << 16)).reshape(
        _FEATURES * _VOCAB, _PDIM)
    return _embed_sum(w_pack, idx)
